# Initial kernel scaffold; baseline (speedup 1.0000x reference)
import jax
import jax.numpy as jnp
from jax.experimental import pallas as pl


def kernel(x, edge_index, W, b):
    raise NotImplementedError("write your pallas kernel here")


# speedup vs baseline: 6.5327x; 6.5327x over previous
import functools

import jax
import jax.numpy as jnp
from jax import lax
from jax.experimental import pallas as pl
from jax.experimental.pallas import tpu as pltpu
from jax.experimental.pallas import tpu_sc as plsc

NC = 2
NS = 16
NW = NC * NS
C = 128
PAD_ROWS = 32
DH = 72
ZR = 128


def _sc_aggregate(x_lo, x_hi, src, dst, n_acc, chunks):
    rpt = n_acc // NS

    mesh = plsc.VectorSubcoreMesh(core_axis_name="c", subcore_axis_name="s")

    @functools.partial(
        pl.kernel,
        mesh=mesh,
        compiler_params=pltpu.CompilerParams(use_tc_tiling_on_sc=False),
        out_type=[
            jax.ShapeDtypeStruct((NC, n_acc, DH), jnp.float32),
            jax.ShapeDtypeStruct((NC, n_acc, DH), jnp.float32),
        ],
        scratch_types=[
            pltpu.VMEM((chunks, C), jnp.int32),
            pltpu.VMEM((chunks, C), jnp.int32),
            pltpu.VMEM((C, DH), jnp.float32),
            pltpu.VMEM((ZR, DH), jnp.float32),
            pltpu.VMEM_SHARED((n_acc, DH), jnp.float32),
            pltpu.SemaphoreType.DMA,
        ],
    )
    def agg_kernel(xlo_hbm, xhi_hbm, src_hbm, dst_hbm, z_hbm,
                   plo_hbm, phi_hbm,
                   src_v, dst_v, rows_v, zbuf, acc, sem):
        cid = lax.axis_index("c")
        sid = lax.axis_index("s")
        wid = cid * NS + sid
        row0 = sid * rpt

        pltpu.sync_copy(z_hbm, zbuf)
        for k in range(rpt // ZR):
            pltpu.sync_copy(zbuf, acc.at[pl.ds(row0 + k * ZR, ZR)])
        pltpu.sync_copy(src_hbm.at[wid], src_v)
        pltpu.sync_copy(dst_hbm.at[wid], dst_v)
        plsc.subcore_barrier()

        def make_body(x_hbm):
            def body(j, carry):
                pltpu.async_copy(x_hbm.at[src_v.at[j]], rows_v, sem).wait()
                pltpu.sync_copy(rows_v, acc.at[dst_v.at[j]], add=True)
                return carry
            return body

        lax.fori_loop(0, chunks, make_body(xlo_hbm), 0)
        plsc.subcore_barrier()
        pltpu.sync_copy(acc.at[pl.ds(row0, rpt)],
                        plo_hbm.at[cid, pl.ds(row0, rpt)])
        for k in range(rpt // ZR):
            pltpu.sync_copy(zbuf, acc.at[pl.ds(row0 + k * ZR, ZR)])
        plsc.subcore_barrier()

        lax.fori_loop(0, chunks, make_body(xhi_hbm), 0)
        plsc.subcore_barrier()
        pltpu.sync_copy(acc.at[pl.ds(row0, rpt)],
                        phi_hbm.at[cid, pl.ds(row0, rpt)])

    zeros = jnp.zeros((ZR, DH), jnp.float32)
    return agg_kernel(x_lo, x_hi, src, dst, zeros)


def _finalize(part_lo, part_hi, W, b, bn):
    n_acc = part_lo.shape[1]
    D = W.shape[0]
    HD = D // 2

    def fin_kernel(pl_ref, ph_ref, w0_ref, w1_ref, b_ref, o_ref):
        s_lo = pl_ref[0] + pl_ref[1]
        s_hi = ph_ref[0] + ph_ref[1]
        deg = jnp.maximum(s_lo[:, HD:HD + 1], 1.0)
        h = jnp.dot(s_lo[:, :HD] / deg, w0_ref[...],
                    preferred_element_type=jnp.float32)
        h += jnp.dot(s_hi[:, :HD] / deg, w1_ref[...],
                     preferred_element_type=jnp.float32)
        o_ref[...] = jnp.maximum(h + b_ref[...], 0.0)

    return pl.pallas_call(
        fin_kernel,
        grid=(n_acc // bn,),
        in_specs=[
            pl.BlockSpec((NC, bn, DH), lambda i: (0, i, 0)),
            pl.BlockSpec((NC, bn, DH), lambda i: (0, i, 0)),
            pl.BlockSpec((HD, D), lambda i: (0, 0)),
            pl.BlockSpec((HD, D), lambda i: (0, 0)),
            pl.BlockSpec((1, D), lambda i: (0, 0)),
        ],
        out_specs=pl.BlockSpec((bn, D), lambda i: (i, 0)),
        out_shape=jax.ShapeDtypeStruct((n_acc, D), jnp.float32),
    )(part_lo, part_hi, W[:HD], W[HD:], b.reshape(1, D))


def kernel(x, edge_index, W, b):
    n_rows, D = x.shape
    E = edge_index.shape[1]
    HD = D // 2

    chunks = -(-E // (NW * C))
    e_pad = NW * chunks * C
    pad = e_pad - E

    ones_col = jnp.ones((n_rows, 1), x.dtype)
    fill = jnp.zeros((n_rows, DH - HD - 1), x.dtype)
    x_lo = jnp.concatenate([x[:, :HD], ones_col, fill], axis=1)
    x_hi = jnp.concatenate([x[:, HD:], ones_col, fill], axis=1)
    x_lo = jnp.pad(x_lo, ((0, PAD_ROWS), (0, 0)))
    x_hi = jnp.pad(x_hi, ((0, PAD_ROWS), (0, 0)))

    pad_ar = jnp.arange(pad, dtype=jnp.int32)
    src = jnp.concatenate(
        [edge_index[0].astype(jnp.int32), n_rows + pad_ar % PAD_ROWS])
    dst = jnp.concatenate(
        [edge_index[1].astype(jnp.int32), pad_ar % n_rows])
    src = src.reshape(NW, chunks, C)
    dst = dst.reshape(NW, chunks, C)

    n_acc = NS * ZR * -(-n_rows // (NS * ZR))
    part_lo, part_hi = _sc_aggregate(x_lo, x_hi, src, dst, n_acc, chunks)
    out = _finalize(part_lo, part_hi, W, b, bn=640)
    return out[:n_rows]


# speedup vs baseline: 8.7396x; 1.3378x over previous
import functools

import jax
import jax.numpy as jnp
from jax import lax
from jax.experimental import pallas as pl
from jax.experimental.pallas import tpu as pltpu
from jax.experimental.pallas import tpu_sc as plsc

NC = 2
NS = 16
C = 128
PAD_ROWS = 32
DH = 72
ZR = 128


def _sc_aggregate(x2, src, dst, n_acc, chunks):
    rpt = n_acc // NS
    nhalf = chunks // 2

    mesh = plsc.VectorSubcoreMesh(core_axis_name="c", subcore_axis_name="s")

    @functools.partial(
        pl.kernel,
        mesh=mesh,
        compiler_params=pltpu.CompilerParams(use_tc_tiling_on_sc=False),
        out_type=jax.ShapeDtypeStruct((NC, n_acc, DH), jnp.float32),
        scratch_types=[
            pltpu.VMEM((chunks, C), jnp.int32),
            pltpu.VMEM((chunks, C), jnp.int32),
            pltpu.VMEM((C, DH), jnp.float32),
            pltpu.VMEM((C, DH), jnp.float32),
            pltpu.VMEM((ZR, DH), jnp.float32),
            pltpu.VMEM_SHARED((n_acc, DH), jnp.float32),
            pltpu.SemaphoreType.DMA,
            pltpu.SemaphoreType.DMA,
        ],
    )
    def agg_kernel(x2_hbm, src_hbm, dst_hbm, z_hbm, part_hbm,
                   src_v, dst_v, rows0, rows1, zbuf, acc, sem0, sem1):
        cid = lax.axis_index("c")
        sid = lax.axis_index("s")
        row0 = sid * rpt
        x_r = x2_hbm.at[cid]

        pltpu.sync_copy(z_hbm, zbuf)
        for k in range(rpt // ZR):
            pltpu.sync_copy(zbuf, acc.at[pl.ds(row0 + k * ZR, ZR)])
        pltpu.sync_copy(src_hbm.at[sid], src_v)
        pltpu.sync_copy(dst_hbm.at[sid], dst_v)
        plsc.subcore_barrier()

        pltpu.async_copy(x_r.at[src_v.at[0]], rows0, sem0)

        def body(i, carry):
            j0 = 2 * i
            pltpu.make_async_copy(x_r.at[src_v.at[j0]], rows0, sem0).wait()
            pltpu.async_copy(x_r.at[src_v.at[j0 + 1]], rows1, sem1)
            pltpu.sync_copy(rows0, acc.at[dst_v.at[j0]], add=True)
            pltpu.make_async_copy(x_r.at[src_v.at[j0 + 1]], rows1, sem1).wait()

            @pl.when(i < nhalf - 1)
            def _():
                pltpu.async_copy(x_r.at[src_v.at[j0 + 2]], rows0, sem0)

            pltpu.sync_copy(rows1, acc.at[dst_v.at[j0 + 1]], add=True)
            return carry

        lax.fori_loop(0, nhalf, body, 0)
        plsc.subcore_barrier()
        pltpu.sync_copy(acc.at[pl.ds(row0, rpt)],
                        part_hbm.at[cid, pl.ds(row0, rpt)])

    zeros = jnp.zeros((ZR, DH), jnp.float32)
    return agg_kernel(x2, src, dst, zeros)


def _finalize(parts, W, b, bn, n_rows):
    D = W.shape[0]
    HD = D // 2

    def fin_kernel(p_ref, w0_ref, w1_ref, b_ref, o_ref):
        s_lo = p_ref[0]
        s_hi = p_ref[1]
        deg = jnp.maximum(s_lo[:, HD:HD + 1], 1.0)
        h = jnp.dot(s_lo[:, :HD] / deg, w0_ref[...],
                    preferred_element_type=jnp.float32)
        h += jnp.dot(s_hi[:, :HD] / deg, w1_ref[...],
                     preferred_element_type=jnp.float32)
        o_ref[...] = jnp.maximum(h + b_ref[...], 0.0)

    return pl.pallas_call(
        fin_kernel,
        grid=(n_rows // bn,),
        in_specs=[
            pl.BlockSpec((NC, bn, DH), lambda i: (0, i, 0)),
            pl.BlockSpec((HD, D), lambda i: (0, 0)),
            pl.BlockSpec((HD, D), lambda i: (0, 0)),
            pl.BlockSpec((1, D), lambda i: (0, 0)),
        ],
        out_specs=pl.BlockSpec((bn, D), lambda i: (i, 0)),
        out_shape=jax.ShapeDtypeStruct((n_rows, D), jnp.float32),
    )(parts, W[:HD], W[HD:], b.reshape(1, D))


def kernel(x, edge_index, W, b):
    n_rows, D = x.shape
    E = edge_index.shape[1]
    HD = D // 2

    chunks = 2 * -(-E // (NS * C * 2))
    e_pad = NS * chunks * C
    pad = e_pad - E

    ones_col = jnp.ones((n_rows, 1), x.dtype)
    fill = jnp.zeros((n_rows, DH - HD - 1), x.dtype)
    x_lo = jnp.concatenate([x[:, :HD], ones_col, fill], axis=1)
    x_hi = jnp.concatenate([x[:, HD:], ones_col, fill], axis=1)
    x2 = jnp.stack([x_lo, x_hi])
    x2 = jnp.pad(x2, ((0, 0), (0, PAD_ROWS), (0, 0)))

    pad_ar = jnp.arange(pad, dtype=jnp.int32)
    src = jnp.concatenate(
        [edge_index[0].astype(jnp.int32), n_rows + pad_ar % PAD_ROWS])
    dst = jnp.concatenate(
        [edge_index[1].astype(jnp.int32), pad_ar % n_rows])
    src = src.reshape(NS, chunks, C)
    dst = dst.reshape(NS, chunks, C)

    n_acc = NS * ZR * -(-n_rows // (NS * ZR))
    parts = _sc_aggregate(x2, src, dst, n_acc, chunks)
    return _finalize(parts, W, b, bn=1000, n_rows=n_rows)


# speedup vs baseline: 11.1108x; 1.2713x over previous
import functools

import jax
import jax.numpy as jnp
from jax import lax
from jax.experimental import pallas as pl
from jax.experimental.pallas import tpu as pltpu
from jax.experimental.pallas import tpu_sc as plsc

NC = 2
NS = 16
C = 128
DH = 72
ZR = 128
RING = 3


def _sc_aggregate(x2, src, dst, n_acc, chunks):
    rpt = n_acc // NS
    steps = chunks // RING

    mesh = plsc.VectorSubcoreMesh(core_axis_name="c", subcore_axis_name="s")

    @functools.partial(
        pl.kernel,
        mesh=mesh,
        compiler_params=pltpu.CompilerParams(use_tc_tiling_on_sc=False),
        out_type=jax.ShapeDtypeStruct((NC, n_acc, DH), jnp.float32),
        scratch_types=[
            pltpu.VMEM((chunks, C), jnp.int32),
            pltpu.VMEM((chunks, C), jnp.int32),
            [pltpu.VMEM((C, DH), jnp.float32) for _ in range(RING)],
            pltpu.VMEM((ZR, DH), jnp.float32),
            pltpu.VMEM_SHARED((n_acc, DH), jnp.float32),
            [pltpu.SemaphoreType.DMA for _ in range(RING)],
            [pltpu.SemaphoreType.DMA for _ in range(RING)],
        ],
    )
    def agg_kernel(x2_hbm, src_hbm, dst_hbm, z_hbm, part_hbm,
                   src_v, dst_v, rows, zbuf, acc, gsem, ssem):
        cid = lax.axis_index("c")
        sid = lax.axis_index("s")
        row0 = sid * rpt
        x_r = x2_hbm.at[cid]

        pltpu.sync_copy(z_hbm, zbuf)
        for k in range(rpt // ZR):
            pltpu.sync_copy(zbuf, acc.at[pl.ds(row0 + k * ZR, ZR)])
        pltpu.sync_copy(src_hbm.at[sid], src_v)
        pltpu.sync_copy(dst_hbm.at[sid], dst_v)
        plsc.subcore_barrier()

        for b in range(RING):
            pltpu.async_copy(x_r.at[src_v.at[b]], rows[b], gsem[b])

        def body(g, carry):
            j0 = RING * g
            scatters = []
            for b in range(RING):
                pltpu.make_async_copy(
                    x_r.at[src_v.at[j0 + b]], rows[b], gsem[b]).wait()
                scatters.append(pltpu.async_copy(
                    rows[b], acc.at[dst_v.at[j0 + b]], ssem[b], add=True))
            for b in range(RING):
                scatters[b].wait()

                @pl.when(g < steps - 1)
                def _():
                    pltpu.async_copy(
                        x_r.at[src_v.at[j0 + RING + b]], rows[b], gsem[b])
            return carry

        lax.fori_loop(0, steps, body, 0)
        plsc.subcore_barrier()
        pltpu.sync_copy(acc.at[pl.ds(row0, rpt)],
                        part_hbm.at[cid, pl.ds(row0, rpt)])

    zeros = jnp.zeros((ZR, DH), jnp.float32)
    return agg_kernel(x2, src, dst, zeros)


def _finalize(parts, W, b, bn, n_rows):
    D = W.shape[0]
    HD = D // 2

    def fin_kernel(p_ref, w0_ref, w1_ref, b_ref, o_ref):
        s_lo = p_ref[0]
        s_hi = p_ref[1]
        deg = jnp.maximum(s_lo[:, HD:HD + 1], 1.0)
        h = jnp.dot(s_lo[:, :HD] / deg, w0_ref[...],
                    preferred_element_type=jnp.float32)
        h += jnp.dot(s_hi[:, :HD] / deg, w1_ref[...],
                     preferred_element_type=jnp.float32)
        o_ref[...] = jnp.maximum(h + b_ref[...], 0.0)

    return pl.pallas_call(
        fin_kernel,
        grid=(n_rows // bn,),
        in_specs=[
            pl.BlockSpec((NC, bn, DH), lambda i: (0, i, 0)),
            pl.BlockSpec((HD, D), lambda i: (0, 0)),
            pl.BlockSpec((HD, D), lambda i: (0, 0)),
            pl.BlockSpec((1, D), lambda i: (0, 0)),
        ],
        out_specs=pl.BlockSpec((bn, D), lambda i: (i, 0)),
        out_shape=jax.ShapeDtypeStruct((n_rows, D), jnp.float32),
    )(parts, W[:HD], W[HD:], b.reshape(1, D))


def kernel(x, edge_index, W, b):
    n_rows, D = x.shape
    E = edge_index.shape[1]
    HD = D // 2

    chunks = RING * -(-E // (NS * C * RING))
    e_pad = NS * chunks * C
    pad = e_pad - E

    ones_col = jnp.ones((n_rows, 1), x.dtype)
    fill = jnp.zeros((n_rows, DH - HD - 1), x.dtype)
    x_lo = jnp.concatenate([x[:, :HD], ones_col, fill], axis=1)
    x_hi = jnp.concatenate([x[:, HD:], ones_col, fill], axis=1)
    x2 = jnp.stack([x_lo, x_hi])

    n_acc = NS * ZR * (n_rows // (NS * ZR) + 1)
    n_junk = n_acc - n_rows

    pad_ar = jnp.arange(pad, dtype=jnp.int32)
    src = jnp.concatenate(
        [edge_index[0].astype(jnp.int32), pad_ar % n_rows])
    dst = jnp.concatenate(
        [edge_index[1].astype(jnp.int32), n_rows + pad_ar % n_junk])
    src = src.reshape(NS, chunks, C)
    dst = dst.reshape(NS, chunks, C)

    parts = _sc_aggregate(x2, src, dst, n_acc, chunks)
    return _finalize(parts, W, b, bn=1000, n_rows=n_rows)


# speedup vs baseline: 11.2108x; 1.0090x over previous
import functools

import jax
import jax.numpy as jnp
from jax import lax
from jax.experimental import pallas as pl
from jax.experimental.pallas import tpu as pltpu
from jax.experimental.pallas import tpu_sc as plsc

NC = 2
NS = 16
C = 128
HD = 64
DH = 72
ZR = 64
RING = 3
BR = 125


def _sc_aggregate(x, src, dst, n_rows, n_acc, chunks):
    rpt = n_acc // NS
    bpt = n_rows // NS
    half = chunks // 2
    steps = half // RING

    mesh = plsc.VectorSubcoreMesh(core_axis_name="c", subcore_axis_name="s")

    @functools.partial(
        pl.kernel,
        mesh=mesh,
        compiler_params=pltpu.CompilerParams(use_tc_tiling_on_sc=False),
        out_type=[
            jax.ShapeDtypeStruct((NC, n_acc, DH), jnp.float32),
            jax.ShapeDtypeStruct((NC, n_rows, DH), jnp.float32),
        ],
        scratch_types=[
            pltpu.VMEM((half, C), jnp.int32),
            pltpu.VMEM((half, C), jnp.int32),
            [pltpu.VMEM((C, DH), jnp.float32) for _ in range(RING)],
            pltpu.VMEM((ZR, DH), jnp.float32),
            pltpu.VMEM((BR, 2 * HD), jnp.float32),
            pltpu.VMEM((BR, DH - HD), jnp.float32),
            pltpu.VMEM_SHARED((n_acc, DH), jnp.float32),
            [pltpu.SemaphoreType.DMA for _ in range(RING)],
            [pltpu.SemaphoreType.DMA for _ in range(RING)],
        ],
    )
    def agg_kernel(x_hbm, src_hbm, dst_hbm, z_hbm, ones_hbm, part_hbm, xh_hbm,
                   src_v, dst_v, rows, zbuf, xbuf, onesbuf, acc,
                   gsem, ssem):
        cid = lax.axis_index("c")
        sid = lax.axis_index("s")
        row0 = sid * rpt
        col0 = cid * HD

        pltpu.sync_copy(ones_hbm, onesbuf)

        def bbody(blk, carry):
            r0 = sid * bpt + blk * BR
            pltpu.sync_copy(x_hbm.at[pl.ds(r0, BR)], xbuf)
            pltpu.sync_copy(xbuf.at[:, pl.ds(col0, HD)],
                            xh_hbm.at[cid, pl.ds(r0, BR), pl.ds(0, HD)])
            pltpu.sync_copy(onesbuf,
                            xh_hbm.at[cid, pl.ds(r0, BR), pl.ds(HD, DH - HD)])
            return carry

        lax.fori_loop(0, bpt // BR, bbody, 0)

        pltpu.sync_copy(z_hbm, zbuf)
        for k in range(rpt // ZR):
            pltpu.sync_copy(zbuf, acc.at[pl.ds(row0 + k * ZR, ZR)])
        plsc.subcore_barrier()

        x_r = xh_hbm.at[cid]

        def sbody(ss, carry):
            pltpu.sync_copy(src_hbm.at[sid, pl.ds(ss * half, half)], src_v)
            pltpu.sync_copy(dst_hbm.at[sid, pl.ds(ss * half, half)], dst_v)
            for b in range(RING):
                pltpu.async_copy(x_r.at[src_v.at[b]], rows[b], gsem[b])

            def body(g, carry):
                j0 = RING * g
                scatters = []
                for b in range(RING):
                    pltpu.make_async_copy(
                        x_r.at[src_v.at[j0 + b]], rows[b], gsem[b]).wait()
                    scatters.append(pltpu.async_copy(
                        rows[b], acc.at[dst_v.at[j0 + b]], ssem[b], add=True))
                for b in range(RING):
                    scatters[b].wait()

                    @pl.when(g < steps - 1)
                    def _():
                        pltpu.async_copy(
                            x_r.at[src_v.at[j0 + RING + b]], rows[b], gsem[b])
                return carry

            lax.fori_loop(0, steps, body, 0)
            return carry

        lax.fori_loop(0, 2, sbody, 0)
        plsc.subcore_barrier()
        pltpu.sync_copy(acc.at[pl.ds(row0, rpt)],
                        part_hbm.at[cid, pl.ds(row0, rpt)])

    zeros = jnp.zeros((ZR, DH), jnp.float32)
    ones_pat = jnp.zeros((BR, DH - HD), jnp.float32).at[:, 0].set(1.0)
    return agg_kernel(x, src, dst, zeros, ones_pat)


def _finalize(parts, W, b, bn, n_rows):
    D = W.shape[0]

    def fin_kernel(p_ref, w0_ref, w1_ref, b_ref, o_ref):
        s_lo = p_ref[0]
        s_hi = p_ref[1]
        deg = jnp.maximum(s_lo[:, HD:HD + 1], 1.0)
        h = jnp.dot(s_lo[:, :HD] / deg, w0_ref[...],
                    preferred_element_type=jnp.float32)
        h += jnp.dot(s_hi[:, :HD] / deg, w1_ref[...],
                     preferred_element_type=jnp.float32)
        o_ref[...] = jnp.maximum(h + b_ref[...], 0.0)

    return pl.pallas_call(
        fin_kernel,
        grid=(n_rows // bn,),
        in_specs=[
            pl.BlockSpec((NC, bn, DH), lambda i: (0, i, 0)),
            pl.BlockSpec((HD, D), lambda i: (0, 0)),
            pl.BlockSpec((HD, D), lambda i: (0, 0)),
            pl.BlockSpec((1, D), lambda i: (0, 0)),
        ],
        out_specs=pl.BlockSpec((bn, D), lambda i: (i, 0)),
        out_shape=jax.ShapeDtypeStruct((n_rows, D), jnp.float32),
    )(parts, W[:HD], W[HD:], b.reshape(1, D))


def kernel(x, edge_index, W, b):
    n_rows, D = x.shape
    E = edge_index.shape[1]

    chunks = 2 * RING * -(-E // (NS * C * 2 * RING))
    e_pad = NS * chunks * C
    pad = e_pad - E

    n_acc = NS * ZR * (n_rows // (NS * ZR) + 1)
    n_junk = n_acc - n_rows

    pad_ar = jnp.arange(pad, dtype=jnp.int32)
    src = jnp.concatenate(
        [edge_index[0].astype(jnp.int32), pad_ar % n_rows])
    dst = jnp.concatenate(
        [edge_index[1].astype(jnp.int32), n_rows + pad_ar % n_junk])
    src = src.reshape(NS, chunks, C)
    dst = dst.reshape(NS, chunks, C)

    parts, _ = _sc_aggregate(x, src, dst, n_rows, n_acc, chunks)
    return _finalize(parts, W, b, bn=1000, n_rows=n_rows)


# speedup vs baseline: 12.1473x; 1.0835x over previous
import functools

import jax
import jax.numpy as jnp
from jax import lax
from jax.experimental import pallas as pl
from jax.experimental.pallas import tpu as pltpu
from jax.experimental.pallas import tpu_sc as plsc

NC = 2
NS = 16
C = 128
HD = 64
DH = 72
ZR = 128
RING = 4
BR = 125
SS = 2


def _sc_aggregate(x, src, dst, n_rows, n_acc, chunks):
    rpt = n_acc // NS
    bpt = n_rows // NS
    nblk = bpt // BR
    half = chunks // SS
    steps = half // RING

    mesh = plsc.VectorSubcoreMesh(core_axis_name="c", subcore_axis_name="s")

    @functools.partial(
        pl.kernel,
        mesh=mesh,
        compiler_params=pltpu.CompilerParams(use_tc_tiling_on_sc=False),
        out_type=[
            jax.ShapeDtypeStruct((NC, n_acc, DH), jnp.float32),
            jax.ShapeDtypeStruct((NC, n_rows, DH), jnp.float32),
        ],
        scratch_types=[
            pltpu.VMEM((half, C), jnp.int32),
            pltpu.VMEM((half, C), jnp.int32),
            [pltpu.VMEM((C, DH), jnp.float32) for _ in range(RING)],
            pltpu.VMEM((ZR, DH), jnp.float32),
            [pltpu.VMEM((BR, HD), jnp.float32) for _ in range(2)],
            pltpu.VMEM((BR, DH - HD), jnp.float32),
            pltpu.VMEM_SHARED((n_acc, DH), jnp.float32),
            [pltpu.SemaphoreType.DMA for _ in range(RING)],
            [pltpu.SemaphoreType.DMA for _ in range(RING)],
            [pltpu.SemaphoreType.DMA for _ in range(2)],
            [pltpu.SemaphoreType.DMA for _ in range(2)],
            pltpu.SemaphoreType.DMA,
        ],
    )
    def agg_kernel(x_hbm, src_hbm, dst_hbm, z_hbm, ones_hbm, part_hbm, xh_hbm,
                   src_v, dst_v, rows, zbuf, xb, onesbuf, acc,
                   gsem, ssem, rsem, wsem, osem):
        cid = lax.axis_index("c")
        sid = lax.axis_index("s")
        row0 = sid * rpt
        col0 = cid * HD

        pltpu.sync_copy(ones_hbm, onesbuf)
        reads, writes, ones_w = [None] * nblk, [None] * nblk, []

        def blk_rows(blk):
            return pl.ds(sid * bpt + blk * BR, BR)

        reads[0] = pltpu.async_copy(
            x_hbm.at[blk_rows(0), pl.ds(col0, HD)], xb[0], rsem[0])
        for blk in range(nblk):
            p = blk % 2
            reads[blk].wait()
            writes[blk] = pltpu.async_copy(
                xb[p], xh_hbm.at[cid, blk_rows(blk), pl.ds(0, HD)], wsem[p])
            ones_w.append(pltpu.async_copy(
                onesbuf, xh_hbm.at[cid, blk_rows(blk), pl.ds(HD, DH - HD)],
                osem))
            if blk + 1 < nblk:
                if blk - 1 >= 0:
                    writes[blk - 1].wait()
                reads[blk + 1] = pltpu.async_copy(
                    x_hbm.at[blk_rows(blk + 1), pl.ds(col0, HD)],
                    xb[(blk + 1) % 2], rsem[(blk + 1) % 2])

        pltpu.sync_copy(z_hbm, zbuf)
        for k in range(rpt // ZR):
            pltpu.sync_copy(zbuf, acc.at[pl.ds(row0 + k * ZR, ZR)])

        writes[nblk - 2].wait()
        writes[nblk - 1].wait()
        for w in ones_w:
            w.wait()
        plsc.subcore_barrier()

        x_r = xh_hbm.at[cid]

        def sbody(ss, carry):
            pltpu.sync_copy(src_hbm.at[sid, pl.ds(ss * half, half)], src_v)
            pltpu.sync_copy(dst_hbm.at[sid, pl.ds(ss * half, half)], dst_v)
            for b in range(RING):
                pltpu.async_copy(x_r.at[src_v.at[b]], rows[b], gsem[b])

            def body(g, carry):
                j0 = RING * g
                scatters = []
                for b in range(RING):
                    pltpu.make_async_copy(
                        x_r.at[src_v.at[j0 + b]], rows[b], gsem[b]).wait()
                    scatters.append(pltpu.async_copy(
                        rows[b], acc.at[dst_v.at[j0 + b]], ssem[b], add=True))
                for b in range(RING):
                    scatters[b].wait()

                    @pl.when(g < steps - 1)
                    def _():
                        pltpu.async_copy(
                            x_r.at[src_v.at[j0 + RING + b]], rows[b], gsem[b])
                return carry

            lax.fori_loop(0, steps, body, 0)
            return carry

        lax.fori_loop(0, SS, sbody, 0)
        plsc.subcore_barrier()
        pltpu.sync_copy(acc.at[pl.ds(row0, rpt)],
                        part_hbm.at[cid, pl.ds(row0, rpt)])

    zeros = jnp.zeros((ZR, DH), jnp.float32)
    ones_pat = jnp.zeros((BR, DH - HD), jnp.float32).at[:, 0].set(1.0)
    return agg_kernel(x, src, dst, zeros, ones_pat)


def _finalize(parts, W, b, bn, n_rows):
    D = W.shape[0]

    def fin_kernel(p_ref, w0_ref, w1_ref, b_ref, o_ref):
        s_lo = p_ref[0]
        s_hi = p_ref[1]
        deg = jnp.maximum(s_lo[:, HD:HD + 1], 1.0)
        h = jnp.dot(s_lo[:, :HD] / deg, w0_ref[...],
                    preferred_element_type=jnp.float32)
        h += jnp.dot(s_hi[:, :HD] / deg, w1_ref[...],
                     preferred_element_type=jnp.float32)
        o_ref[...] = jnp.maximum(h + b_ref[...], 0.0)

    return pl.pallas_call(
        fin_kernel,
        grid=(n_rows // bn,),
        in_specs=[
            pl.BlockSpec((NC, bn, DH), lambda i: (0, i, 0)),
            pl.BlockSpec((HD, D), lambda i: (0, 0)),
            pl.BlockSpec((HD, D), lambda i: (0, 0)),
            pl.BlockSpec((1, D), lambda i: (0, 0)),
        ],
        out_specs=pl.BlockSpec((bn, D), lambda i: (i, 0)),
        out_shape=jax.ShapeDtypeStruct((n_rows, D), jnp.float32),
    )(parts, W[:HD], W[HD:], b.reshape(1, D))


def kernel(x, edge_index, W, b):
    n_rows, D = x.shape
    E = edge_index.shape[1]

    chunks = SS * RING * -(-E // (NS * C * SS * RING))
    e_pad = NS * chunks * C
    pad = e_pad - E

    n_acc = NS * ZR * (n_rows // (NS * ZR) + 1)
    n_junk = n_acc - n_rows

    pad_ar = jnp.arange(pad, dtype=jnp.int32)
    src = jnp.concatenate(
        [edge_index[0].astype(jnp.int32), pad_ar % n_rows])
    dst = jnp.concatenate(
        [edge_index[1].astype(jnp.int32), n_rows + pad_ar % n_junk])
    src = src.reshape(NS, chunks, C)
    dst = dst.reshape(NS, chunks, C)

    parts, _ = _sc_aggregate(x, src, dst, n_rows, n_acc, chunks)
    return _finalize(parts, W, b, bn=1000, n_rows=n_rows)
